# Initial kernel scaffold; baseline (speedup 1.0000x reference)
import jax
import jax.numpy as jnp
from jax.experimental import pallas as pl


def kernel(x, h_table, w_table):
    raise NotImplementedError("write your pallas kernel here")


# speedup vs baseline: 1.0479x; 1.0479x over previous
import jax
import jax.numpy as jnp
from jax.experimental import pallas as pl


def _add_pos_kernel(x_ref, h_ref, w_ref, o_ref):
    h = h_ref[...][:, :, None, :]
    w = w_ref[...][:, None, :, :]
    o_ref[...] = x_ref[...] + h + w


def kernel(x, h_table, w_table):
    B, H, W, D = x.shape
    grid = (B,)
    return pl.pallas_call(
        _add_pos_kernel,
        grid=grid,
        in_specs=[
            pl.BlockSpec((1, H, W, D), lambda b: (b, 0, 0, 0)),
            pl.BlockSpec((1, H, D), lambda b: (0, 0, 0)),
            pl.BlockSpec((1, W, D), lambda b: (0, 0, 0)),
        ],
        out_specs=pl.BlockSpec((1, H, W, D), lambda b: (b, 0, 0, 0)),
        out_shape=jax.ShapeDtypeStruct((B, H, W, D), x.dtype),
    )(x, h_table[None], w_table[None])
